# baseline (speedup 1.0000x reference)
import jax
import jax.numpy as jnp
from jax import lax
from jax.experimental import pallas as pl
from jax.experimental.pallas import tpu as pltpu
from jax.experimental.pallas import tpu_sc as plsc

BATCH = 16384
ENT_DIM = 128
HALF = 64
NUM_WORKERS = 32
SAMPLES_PER_WORKER = BATCH // NUM_WORKERS
CHUNK = 128
NUM_CHUNKS = SAMPLES_PER_WORKER // CHUNK
LANES = 16
GROUPS_PER_CHUNK = CHUNK // LANES


def _body(hidx_hbm, ridx_hbm, tidx_hbm, ent_hbm, rel_hbm, out_hbm,
          hidx_v, ridx_v, tidx_v, h_buf, r_buf, t_buf, score_v, sem):
    wid = lax.axis_index("s") * 2 + lax.axis_index("c")
    wbase = wid * SAMPLES_PER_WORKER
    iota = lax.broadcasted_iota(jnp.int32, (LANES,), 0)

    def chunk_body(c, carry):
        base = wbase + c * CHUNK
        pltpu.sync_copy(hidx_hbm.at[pl.ds(base, CHUNK)], hidx_v)
        pltpu.sync_copy(ridx_hbm.at[pl.ds(base, CHUNK)], ridx_v)
        pltpu.sync_copy(tidx_hbm.at[pl.ds(base, CHUNK)], tidx_v)
        ch = pltpu.async_copy(ent_hbm.at[hidx_v], h_buf, sem)
        cr = pltpu.async_copy(rel_hbm.at[ridx_v], r_buf, sem)
        ct = pltpu.async_copy(ent_hbm.at[tidx_v], t_buf, sem)
        ch.wait()
        cr.wait()
        ct.wait()

        def group_body(g, carry2):
            row_idx = g * LANES + iota
            acc = jnp.zeros((LANES,), jnp.float32)
            for d in range(HALF):
                re_col = jnp.full((LANES,), d, jnp.int32)
                im_col = jnp.full((LANES,), d + HALF, jnp.int32)
                rh = plsc.load_gather(h_buf, [row_idx, re_col])
                ih = plsc.load_gather(h_buf, [row_idx, im_col])
                rr = plsc.load_gather(r_buf, [row_idx, re_col])
                ir = plsc.load_gather(r_buf, [row_idx, im_col])
                rt = plsc.load_gather(t_buf, [row_idx, re_col])
                it = plsc.load_gather(t_buf, [row_idx, im_col])
                acc = acc + (rh * rr - ih * ir) * rt + (rh * ir + ih * rr) * it
            score_v[pl.ds(c * CHUNK + g * LANES, LANES)] = acc
            return carry2

        lax.fori_loop(0, GROUPS_PER_CHUNK, group_body, 0)
        return carry

    lax.fori_loop(0, NUM_CHUNKS, chunk_body, 0)
    pltpu.sync_copy(score_v, out_hbm.at[pl.ds(wbase, SAMPLES_PER_WORKER)])


_sc_call = pl.kernel(
    _body,
    out_type=jax.ShapeDtypeStruct((BATCH,), jnp.float32),
    mesh=plsc.VectorSubcoreMesh(core_axis_name="c", subcore_axis_name="s"),
    scratch_types=[
        pltpu.VMEM((CHUNK,), jnp.int32),
        pltpu.VMEM((CHUNK,), jnp.int32),
        pltpu.VMEM((CHUNK,), jnp.int32),
        pltpu.VMEM((CHUNK, ENT_DIM), jnp.float32),
        pltpu.VMEM((CHUNK, ENT_DIM), jnp.float32),
        pltpu.VMEM((CHUNK, ENT_DIM), jnp.float32),
        pltpu.VMEM((SAMPLES_PER_WORKER,), jnp.float32),
        pltpu.SemaphoreType.DMA,
    ],
    compiler_params=pltpu.CompilerParams(needs_layout_passes=False),
)


@jax.jit
def kernel(sample, entity_embedding, relation_embedding):
    hidx = sample[:, 0]
    ridx = sample[:, 1]
    tidx = sample[:, 2]
    score = _sc_call(hidx, ridx, tidx, entity_embedding, relation_embedding)
    return score.reshape(BATCH, 1), jnp.zeros((), dtype=jnp.float32)


# speedup vs baseline: 2.8991x; 2.8991x over previous
import jax
import jax.numpy as jnp
from jax import lax
from jax.experimental import pallas as pl
from jax.experimental.pallas import tpu as pltpu
from jax.experimental.pallas import tpu_sc as plsc

BATCH = 16384
ENT_DIM = 128
HALF = 64
LANES = 16
NUM_WORKERS = 32
SAMPLES_PER_WORKER = BATCH // NUM_WORKERS
CHUNK = 128
NUM_CHUNKS = SAMPLES_PER_WORKER // CHUNK


def _body(hidx_hbm, ridx_hbm, tidx_hbm, ent_hbm, rel_hbm, out_hbm,
          hidx_v, ridx_v, tidx_v,
          h_bufs, r_bufs, t_bufs, score_v, sem0, sem1):
    wid = lax.axis_index("s") * 2 + lax.axis_index("c")
    wbase = wid * SAMPLES_PER_WORKER
    iota = lax.broadcasted_iota(jnp.int32, (LANES,), 0)
    last_lane = iota == (LANES - 1)
    sems = (sem0, sem1)

    for c in range(NUM_CHUNKS):
        base = wbase + c * CHUNK
        pltpu.sync_copy(hidx_hbm.at[pl.ds(base, CHUNK)], hidx_v.at[c])
        pltpu.sync_copy(ridx_hbm.at[pl.ds(base, CHUNK)], ridx_v.at[c])
        pltpu.sync_copy(tidx_hbm.at[pl.ds(base, CHUNK)], tidx_v.at[c])

    def start_gathers(c):
        par = c % 2
        s = sems[par]
        return (pltpu.async_copy(ent_hbm.at[hidx_v.at[c]], h_bufs[par], s),
                pltpu.async_copy(rel_hbm.at[ridx_v.at[c]], r_bufs[par], s),
                pltpu.async_copy(ent_hbm.at[tidx_v.at[c]], t_bufs[par], s))

    inflight = start_gathers(0)

    for c in range(NUM_CHUNKS):
        par = c % 2
        for cp in inflight:
            cp.wait()
        if c + 1 < NUM_CHUNKS:
            inflight = start_gathers(c + 1)
        h_buf, r_buf, t_buf = h_bufs[par], r_bufs[par], t_bufs[par]

        def sample_body(s, carry):
            acc = jnp.zeros((LANES,), jnp.float32)
            for k in range(HALF // LANES):
                re_sl = pl.ds(k * LANES, LANES)
                im_sl = pl.ds(HALF + k * LANES, LANES)
                rh = h_buf[s, re_sl]
                ih = h_buf[s, im_sl]
                rr = r_buf[s, re_sl]
                ir = r_buf[s, im_sl]
                rt = t_buf[s, re_sl]
                it = t_buf[s, im_sl]
                acc = acc + (rh * rr - ih * ir) * rt + (rh * ir + ih * rr) * it
            cum = plsc.cumsum(acc)
            pos = jnp.full((LANES,), carry + s, jnp.int32)
            plsc.store_scatter(score_v, [pos], cum, mask=last_lane)
            return carry

        lax.fori_loop(0, CHUNK, sample_body, c * CHUNK)

    pltpu.sync_copy(score_v, out_hbm.at[pl.ds(wbase, SAMPLES_PER_WORKER)])


_sc_call = pl.kernel(
    _body,
    out_type=jax.ShapeDtypeStruct((BATCH,), jnp.float32),
    mesh=plsc.VectorSubcoreMesh(core_axis_name="c", subcore_axis_name="s"),
    scratch_types=[
        pltpu.VMEM((NUM_CHUNKS, CHUNK), jnp.int32),
        pltpu.VMEM((NUM_CHUNKS, CHUNK), jnp.int32),
        pltpu.VMEM((NUM_CHUNKS, CHUNK), jnp.int32),
        (pltpu.VMEM((CHUNK, ENT_DIM), jnp.float32),
         pltpu.VMEM((CHUNK, ENT_DIM), jnp.float32)),
        (pltpu.VMEM((CHUNK, ENT_DIM), jnp.float32),
         pltpu.VMEM((CHUNK, ENT_DIM), jnp.float32)),
        (pltpu.VMEM((CHUNK, ENT_DIM), jnp.float32),
         pltpu.VMEM((CHUNK, ENT_DIM), jnp.float32)),
        pltpu.VMEM((SAMPLES_PER_WORKER,), jnp.float32),
        pltpu.SemaphoreType.DMA,
        pltpu.SemaphoreType.DMA,
    ],
    compiler_params=pltpu.CompilerParams(needs_layout_passes=False),
)


@jax.jit
def kernel(sample, entity_embedding, relation_embedding):
    hidx = sample[:, 0]
    ridx = sample[:, 1]
    tidx = sample[:, 2]
    score = _sc_call(hidx, ridx, tidx, entity_embedding, relation_embedding)
    return score.reshape(BATCH, 1), jnp.zeros((), dtype=jnp.float32)
